# Initial kernel scaffold; baseline (speedup 1.0000x reference)
import jax
import jax.numpy as jnp
from jax.experimental import pallas as pl


def kernel(inp, adj, W, a):
    raise NotImplementedError("write your pallas kernel here")


# speedup vs baseline: 1.3021x; 1.3021x over previous
import jax
import jax.numpy as jnp
from jax import lax
from jax.experimental import pallas as pl
from jax.experimental.pallas import tpu as pltpu

_B, _N, _INF, _OUTF = 4, 1024, 256, 256


def _gat_body(inp_ref, adjm_ref, w_ref, at_ref, out_ref):
    x = inp_ref[0]
    h = jnp.dot(x, w_ref[...], preferred_element_type=jnp.float32)
    row = lax.broadcasted_iota(jnp.int32, (_N, 1), 0)
    h = jnp.where(row == 0, 0.0, h)
    m = (adjm_ref[...] > 0).astype(jnp.float32)
    g = jnp.dot(m, h, preferred_element_type=jnp.float32)
    at = at_ref[...]
    vals = (jnp.sum(h * at[:, :_OUTF], axis=1, keepdims=True)
            + jnp.sum(g * at[:, _OUTF:], axis=1, keepdims=True))
    vals = jnp.where(row == 0, 0.0, vals)
    s = jnp.sum(h, axis=0, keepdims=True)
    o = vals * s
    out_ref[0] = jnp.maximum(o, 0.2 * o)


def kernel(inp, adj, W, a):
    adjm = jnp.pad(adj.T[:, 1:], ((0, 0), (0, 1)))
    at = a.T
    return pl.pallas_call(
        _gat_body,
        grid=(_B,),
        in_specs=[
            pl.BlockSpec((1, _N, _INF), lambda b: (b, 0, 0)),
            pl.BlockSpec((_N, _N), lambda b: (0, 0)),
            pl.BlockSpec((_INF, _OUTF), lambda b: (0, 0)),
            pl.BlockSpec((_N, 2 * _OUTF), lambda b: (0, 0)),
        ],
        out_specs=pl.BlockSpec((1, _N, _OUTF), lambda b: (b, 0, 0)),
        out_shape=jax.ShapeDtypeStruct((_B, _N, _OUTF), jnp.float32),
        compiler_params=pltpu.CompilerParams(
            dimension_semantics=("arbitrary",),
        ),
    )(inp, adjm, W, at)


# speedup vs baseline: 2.4685x; 1.8958x over previous
import jax
import jax.numpy as jnp
from jax import lax
from jax.experimental import pallas as pl
from jax.experimental.pallas import tpu as pltpu

_B, _N, _INF, _OUTF = 4, 1024, 256, 256


def _gat_body(inp_ref, adj_ref, w_ref, a_ref, out_ref, at_s):
    @pl.when(pl.program_id(0) == 0)
    def _():
        at_s[...] = jnp.transpose(a_ref[...])

    x = inp_ref[0]
    h = jnp.dot(x, w_ref[...], preferred_element_type=jnp.float32)
    row = lax.broadcasted_iota(jnp.int32, (_N, 1), 0)
    h = jnp.where(row == 0, 0.0, h)
    hp = pltpu.roll(h, 1, 0)
    hp = jnp.where(row == 0, 0.0, hp)
    m = (adj_ref[...] > 0).astype(jnp.float32)
    g = lax.dot_general(m, hp, (((0,), (0,)), ((), ())),
                        preferred_element_type=jnp.float32)
    at = at_s[...]
    vals = (jnp.sum(h * at[:, :_OUTF], axis=1, keepdims=True)
            + jnp.sum(g * at[:, _OUTF:], axis=1, keepdims=True))
    vals = jnp.where(row == 0, 0.0, vals)
    s = jnp.sum(h, axis=0, keepdims=True)
    o = vals * s
    out_ref[0] = jnp.maximum(o, 0.2 * o)


def kernel(inp, adj, W, a):
    return pl.pallas_call(
        _gat_body,
        grid=(_B,),
        in_specs=[
            pl.BlockSpec((1, _N, _INF), lambda b: (b, 0, 0)),
            pl.BlockSpec((_N, _N), lambda b: (0, 0)),
            pl.BlockSpec((_INF, _OUTF), lambda b: (0, 0)),
            pl.BlockSpec((2 * _OUTF, _N), lambda b: (0, 0)),
        ],
        out_specs=pl.BlockSpec((1, _N, _OUTF), lambda b: (b, 0, 0)),
        out_shape=jax.ShapeDtypeStruct((_B, _N, _OUTF), jnp.float32),
        scratch_shapes=[pltpu.VMEM((_N, 2 * _OUTF), jnp.float32)],
        compiler_params=pltpu.CompilerParams(
            dimension_semantics=("arbitrary",),
        ),
    )(inp, adj, W, a)
